# Initial kernel scaffold; baseline (speedup 1.0000x reference)
import jax
import jax.numpy as jnp
from jax.experimental import pallas as pl


def kernel(x, edge_index, edge_weight, W_ln1, b_ln1, W_l1, W_r1, b1, W_ln2, b_ln2, W_l2, W_r2, b2):
    raise NotImplementedError("write your pallas kernel here")


# speedup vs baseline: 1.0313x; 1.0313x over previous
import jax
import jax.numpy as jnp
from jax.experimental import pallas as pl


def _final_add(a_ref, b_ref, o_ref):
    o_ref[...] = a_ref[...] + b_ref[...]


def _sage(h, edge_index, ew, Wl, Wr, b):
    src = edge_index[0]
    dst = edge_index[1]
    msg = h[src] * ew[:, None]
    s = jax.ops.segment_sum(msg, dst, num_segments=h.shape[0])
    denom = jax.ops.segment_sum(ew, dst, num_segments=h.shape[0])
    agg = s / jnp.maximum(denom, 1e-6)[:, None]
    return agg @ Wl + h @ Wr + b


def kernel(x, edge_index, edge_weight, W_ln1, b_ln1, W_l1, W_r1, b1,
           W_ln2, b_ln2, W_l2, W_r2, b2):
    lx1 = jax.nn.relu(x @ W_ln1 + b_ln1)
    rst1 = _sage(lx1, edge_index, edge_weight, W_l1, W_r1, b1)
    lx2 = jax.nn.relu(rst1 @ W_ln2 + b_ln2)
    src = edge_index[0]
    dst = edge_index[1]
    g2 = lx2 @ W_l2
    msg = g2[src] * edge_weight[:, None]
    s = jax.ops.segment_sum(msg, dst, num_segments=lx2.shape[0])
    denom = jax.ops.segment_sum(edge_weight, dst, num_segments=lx2.shape[0])
    t = s / jnp.maximum(denom, 1e-6)[:, None]
    r = lx2 @ W_r2 + b2
    return pl.pallas_call(
        _final_add,
        out_shape=jax.ShapeDtypeStruct(t.shape, t.dtype),
    )(t, r)


# speedup vs baseline: 3.5921x; 3.4831x over previous
import jax
import jax.numpy as jnp
from jax import lax
from jax.experimental import pallas as pl
from jax.experimental.pallas import tpu as pltpu
from jax.experimental.pallas import tpu_sc as plsc

_NC = 2
_NS = 16
_B = 80
_ZB = 128
_RPAD = _NS * _ZB


def _edge_body_den(src_h, dst_h, ew_h, g_h, out_h, den_h,
                   src_v, dst_v, ew_v, rows_v, zacc_v, ewb_v, zden_v,
                   acc_sh, den_sh, sem):
    _edge_common(src_h, dst_h, ew_h, g_h, out_h, den_h,
                 src_v, dst_v, ew_v, rows_v, zacc_v, ewb_v, zden_v,
                 acc_sh, den_sh, sem)


def _edge_body_noden(src_h, dst_h, ew_h, g_h, out_h,
                     src_v, dst_v, ew_v, rows_v, zacc_v, acc_sh, sem):
    _edge_common(src_h, dst_h, ew_h, g_h, out_h, None,
                 src_v, dst_v, ew_v, rows_v, zacc_v, None, None,
                 acc_sh, None, sem)


def _edge_common(src_h, dst_h, ew_h, g_h, out_h, den_h,
                 src_v, dst_v, ew_v, rows_v, zacc_v, ewb_v, zden_v,
                 acc_sh, den_sh, sem):
    n_pad, wd = acc_sh.shape
    ch, b = src_v.shape
    rps = n_pad // _NS
    zn = rps // _ZB
    c = lax.axis_index("c")
    s = lax.axis_index("s")

    def zrow(i, carry):
        for cc in range(wd // 16):
            zacc_v[i, pl.ds(cc * 16, 16)] = jnp.zeros((16,), jnp.float32)
        if zden_v is not None:
            zden_v[i, :] = jnp.zeros((16,), jnp.float32)
        return carry

    lax.fori_loop(0, _ZB, zrow, 0)
    for z in range(zn):
        row0 = s * rps + z * _ZB
        pltpu.sync_copy(zacc_v, acc_sh.at[pl.ds(row0, _ZB)])
        if den_sh is not None:
            pltpu.sync_copy(zden_v, den_sh.at[pl.ds(row0, _ZB)])
    plsc.subcore_barrier()

    pltpu.sync_copy(src_h.at[s], src_v)
    pltpu.sync_copy(dst_h.at[s], dst_v)
    pltpu.sync_copy(ew_h.at[s], ew_v)

    def chunk(j, carry):
        pltpu.async_copy(g_h.at[c].at[src_v.at[j]], rows_v, sem).wait()

        def group(g, carry2):
            ew16 = ew_v[j, pl.ds(g * 16, 16)]
            for i in range(16):
                ebase = g * 16 + i
                wv = jnp.full((16,), ew16[i], jnp.float32)
                for cc in range(wd // 16):
                    sl = pl.ds(cc * 16, 16)
                    rows_v[ebase, sl] = rows_v[ebase, sl] * wv
                if ewb_v is not None:
                    ewb_v[ebase, :] = wv
            return carry2

        lax.fori_loop(0, b // 16, group, 0)
        pltpu.sync_copy(rows_v, acc_sh.at[dst_v.at[j]], add=True)
        if den_sh is not None:
            @pl.when(c == 0)
            def _den_add():
                pltpu.sync_copy(ewb_v, den_sh.at[dst_v.at[j]], add=True)
        return carry

    lax.fori_loop(0, ch, chunk, 0)
    plsc.subcore_barrier()

    row0 = s * rps
    pltpu.sync_copy(acc_sh.at[pl.ds(row0, rps)], out_h.at[c, pl.ds(row0, rps)])
    if den_sh is not None:
        @pl.when(c == 0)
        def _den_out():
            pltpu.sync_copy(den_sh.at[pl.ds(row0, rps)],
                            den_h.at[pl.ds(row0, rps)])


def _make_edge_call(n_pad, wd, ch, with_den):
    mesh = plsc.VectorSubcoreMesh(core_axis_name="c", subcore_axis_name="s",
                                  num_cores=_NC, num_subcores=_NS)
    out_type = [jax.ShapeDtypeStruct((_NC, n_pad, wd), jnp.float32)]
    scratch = [
        pltpu.VMEM((ch, _B), jnp.int32),
        pltpu.VMEM((ch, _B), jnp.int32),
        pltpu.VMEM((ch, _B), jnp.float32),
        pltpu.VMEM((_B, wd), jnp.float32),
        pltpu.VMEM((_ZB, wd), jnp.float32),
    ]
    if with_den:
        out_type.append(jax.ShapeDtypeStruct((n_pad, 16), jnp.float32))
        scratch += [pltpu.VMEM((_B, 16), jnp.float32),
                    pltpu.VMEM((_ZB, 16), jnp.float32)]
    scratch.append(pltpu.VMEM_SHARED((n_pad, wd), jnp.float32))
    if with_den:
        scratch.append(pltpu.VMEM_SHARED((n_pad, 16), jnp.float32))
    scratch.append(pltpu.SemaphoreType.DMA)
    body = _edge_body_den if with_den else _edge_body_noden
    return pl.kernel(body, out_type=tuple(out_type), mesh=mesh,
                     scratch_types=scratch,
                     compiler_params=pltpu.CompilerParams(
                         use_tc_tiling_on_sc=False))


def _tc_pre_body(x_ref, wln_ref, bln_ref, wla_ref, wlb_ref, wr_ref, b_ref,
                 ga_ref, gb_ref, r_ref):
    lx = jnp.maximum(
        jnp.dot(x_ref[...], wln_ref[...], preferred_element_type=jnp.float32)
        + bln_ref[...], 0.0)
    ga_ref[...] = jnp.dot(lx, wla_ref[...], preferred_element_type=jnp.float32)
    gb_ref[...] = jnp.dot(lx, wlb_ref[...], preferred_element_type=jnp.float32)
    r_ref[...] = jnp.dot(lx, wr_ref[...],
                         preferred_element_type=jnp.float32) + b_ref[...]


def _tc_mid_body(ta_ref, tb_ref, d_ref, r_ref,
                 wln_ref, bln_ref, wla_ref, wlb_ref, wr_ref, b_ref,
                 ga_ref, gb_ref, r2_ref):
    den = jnp.maximum(d_ref[...][:, :1], 1e-6)
    t = jnp.concatenate([ta_ref[...], tb_ref[...]], axis=1)
    rst = t / den + r_ref[...]
    lx = jnp.maximum(
        jnp.dot(rst, wln_ref[...], preferred_element_type=jnp.float32)
        + bln_ref[...], 0.0)
    ga_ref[...] = jnp.dot(lx, wla_ref[...], preferred_element_type=jnp.float32)
    gb_ref[...] = jnp.dot(lx, wlb_ref[...], preferred_element_type=jnp.float32)
    r2_ref[...] = jnp.dot(lx, wr_ref[...],
                          preferred_element_type=jnp.float32) + b_ref[...]


def _tc_post_body(ta_ref, tb_ref, d_ref, r_ref, o_ref):
    den = jnp.maximum(d_ref[...][:, :1], 1e-6)
    t = jnp.concatenate([ta_ref[...], tb_ref[...]], axis=1)
    o_ref[...] = t / den + r_ref[...]


def _row_spec(rb, w):
    return pl.BlockSpec((rb, w), lambda i: (i, 0))


def _full_spec(shape):
    return pl.BlockSpec(shape, lambda i: tuple(0 for _ in shape))


def kernel(x, edge_index, edge_weight, W_ln1, b_ln1, W_l1, W_r1, b1,
           W_ln2, b_ln2, W_l2, W_r2, b2):
    n, d = x.shape
    h = W_l1.shape[1]
    cdim = W_l2.shape[1]
    e = edge_weight.shape[0]
    eps = e // _NS
    ch = eps // _B
    rb = 2000
    grid = (n // rb,)
    n_pad = ((n + _RPAD - 1) // _RPAD) * _RPAD

    src = edge_index[0].reshape(_NS, ch, _B)
    dst = edge_index[1].reshape(_NS, ch, _B)
    ewr = edge_weight.reshape(_NS, ch, _B)

    g1a, g1b, r1 = pl.pallas_call(
        _tc_pre_body,
        grid=grid,
        in_specs=[_row_spec(rb, d), _full_spec((d, d)), _full_spec((1, d)),
                  _full_spec((d, h // 2)), _full_spec((d, h // 2)),
                  _full_spec((d, h)), _full_spec((1, h))],
        out_specs=[_row_spec(rb, h // 2), _row_spec(rb, h // 2),
                   _row_spec(rb, h)],
        out_shape=[jax.ShapeDtypeStruct((n, h // 2), jnp.float32),
                   jax.ShapeDtypeStruct((n, h // 2), jnp.float32),
                   jax.ShapeDtypeStruct((n, h), jnp.float32)],
    )(x, W_ln1, b_ln1.reshape(1, d), W_l1[:, :h // 2], W_l1[:, h // 2:],
      W_r1, b1.reshape(1, h))

    g1s = jnp.stack([g1a, g1b])
    t1p, denp = _make_edge_call(n_pad, h // 2, ch, True)(src, dst, ewr, g1s)

    g2a, g2b, r2 = pl.pallas_call(
        _tc_mid_body,
        grid=grid,
        in_specs=[_row_spec(rb, h // 2), _row_spec(rb, h // 2),
                  _row_spec(rb, 16), _row_spec(rb, h),
                  _full_spec((h, h)), _full_spec((1, h)),
                  _full_spec((h, cdim // 2)), _full_spec((h, cdim // 2)),
                  _full_spec((h, cdim)), _full_spec((1, cdim))],
        out_specs=[_row_spec(rb, cdim // 2), _row_spec(rb, cdim // 2),
                   _row_spec(rb, cdim)],
        out_shape=[jax.ShapeDtypeStruct((n, cdim // 2), jnp.float32),
                   jax.ShapeDtypeStruct((n, cdim // 2), jnp.float32),
                   jax.ShapeDtypeStruct((n, cdim), jnp.float32)],
    )(t1p[0, :n], t1p[1, :n], denp[:n], r1, W_ln2, b_ln2.reshape(1, h),
      W_l2[:, :cdim // 2], W_l2[:, cdim // 2:], W_r2, b2.reshape(1, cdim))

    g2s = jnp.stack([g2a, g2b])
    (t2p,) = _make_edge_call(n_pad, cdim // 2, ch, False)(src, dst, ewr, g2s)

    out = pl.pallas_call(
        _tc_post_body,
        grid=grid,
        in_specs=[_row_spec(rb, cdim // 2), _row_spec(rb, cdim // 2),
                  _row_spec(rb, 16), _row_spec(rb, cdim)],
        out_specs=_row_spec(rb, cdim),
        out_shape=jax.ShapeDtypeStruct((n, cdim), jnp.float32),
    )(t2p[0, :n], t2p[1, :n], denp[:n], r2)
    return out


# speedup vs baseline: 3.7129x; 1.0336x over previous
import jax
import jax.numpy as jnp
from jax import lax
from jax.experimental import pallas as pl
from jax.experimental.pallas import tpu as pltpu
from jax.experimental.pallas import tpu_sc as plsc

_NC = 2
_NS = 16
_B = 80
_ZB = 128
_RPAD = _NS * _ZB


def _edge_body_den(src_h, dst_h, ewb_h, g_h, out_h, den_h,
                   src_v, dst_v, rows_a, rows_b, ewc_a, ewc_b,
                   zacc_v, zden_v, acc_sh, den_sh, sga, sgb, ssa, ssb):
    _edge_common(src_h, dst_h, ewb_h, g_h, out_h, den_h,
                 src_v, dst_v, rows_a, rows_b, ewc_a, ewc_b,
                 zacc_v, zden_v, acc_sh, den_sh, sga, sgb, ssa, ssb)


def _edge_body_noden(src_h, dst_h, ewb_h, g_h, out_h,
                     src_v, dst_v, rows_a, rows_b, ewc_a, ewc_b,
                     zacc_v, acc_sh, sga, sgb, ssa, ssb):
    _edge_common(src_h, dst_h, ewb_h, g_h, out_h, None,
                 src_v, dst_v, rows_a, rows_b, ewc_a, ewc_b,
                 zacc_v, None, acc_sh, None, sga, sgb, ssa, ssb)


def _edge_common(src_h, dst_h, ewb_h, g_h, out_h, den_h,
                 src_v, dst_v, rows_a, rows_b, ewc_a, ewc_b,
                 zacc_v, zden_v, acc_sh, den_sh, sga, sgb, ssa, ssb):
    n_pad, wd = acc_sh.shape
    ch, b = src_v.shape
    rps = n_pad // _NS
    zn = rps // _ZB
    c = lax.axis_index("c")
    s = lax.axis_index("s")

    def zrow(i, carry):
        for cc in range(wd // 16):
            zacc_v[i, pl.ds(cc * 16, 16)] = jnp.zeros((16,), jnp.float32)
        if zden_v is not None:
            zden_v[i, :] = jnp.zeros((16,), jnp.float32)
        return carry

    lax.fori_loop(0, _ZB, zrow, 0)
    for z in range(zn):
        row0 = s * rps + z * _ZB
        pltpu.sync_copy(zacc_v, acc_sh.at[pl.ds(row0, _ZB)])
        if den_sh is not None:
            pltpu.sync_copy(zden_v, den_sh.at[pl.ds(row0, _ZB)])
    plsc.subcore_barrier()

    pltpu.sync_copy(src_h.at[s], src_v)
    pltpu.sync_copy(dst_h.at[s], dst_v)

    def start_gather(j, rows_v, ewc_v, sem):
        pltpu.async_copy(g_h.at[c].at[src_v.at[j]], rows_v, sem)
        pltpu.async_copy(ewb_h.at[s, j], ewc_v, sem)

    def wait_gather(j, rows_v, ewc_v, sem):
        pltpu.make_async_copy(g_h.at[c].at[src_v.at[j]], rows_v, sem).wait()
        pltpu.make_async_copy(ewb_h.at[s, j], ewc_v, sem).wait()

    def scale(rows_v, ewc_v):
        def grp(g8, carry):
            for i in range(8):
                e = g8 * 8 + i
                wv = ewc_v[e, :]
                for cc in range(wd // 16):
                    sl = pl.ds(cc * 16, 16)
                    rows_v[e, sl] = rows_v[e, sl] * wv
            return carry
        lax.fori_loop(0, b // 8, grp, 0)

    def proc(j, rows_v, ewc_v, sem):
        scale(rows_v, ewc_v)
        d = pltpu.async_copy(rows_v, acc_sh.at[dst_v.at[j]], sem, add=True)
        if den_sh is not None:
            @pl.when(c == 0)
            def _():
                pltpu.sync_copy(ewc_v, den_sh.at[dst_v.at[j]], add=True)
        return d

    start_gather(0, rows_a, ewc_a, sga)
    start_gather(1, rows_b, ewc_b, sgb)

    def pair(p, carry):
        j = 2 * p
        wait_gather(j, rows_a, ewc_a, sga)
        da = proc(j, rows_a, ewc_a, ssa)
        wait_gather(j + 1, rows_b, ewc_b, sgb)
        db = proc(j + 1, rows_b, ewc_b, ssb)
        da.wait()

        @pl.when(j + 2 < ch)
        def _():
            start_gather(j + 2, rows_a, ewc_a, sga)

        db.wait()

        @pl.when(j + 3 < ch)
        def _():
            start_gather(j + 3, rows_b, ewc_b, sgb)

        return carry

    lax.fori_loop(0, ch // 2, pair, 0)
    plsc.subcore_barrier()

    row0 = s * rps
    pltpu.sync_copy(acc_sh.at[pl.ds(row0, rps)], out_h.at[c, pl.ds(row0, rps)])
    if den_sh is not None:
        @pl.when(c == 0)
        def _():
            pltpu.sync_copy(den_sh.at[pl.ds(row0, rps)],
                            den_h.at[pl.ds(row0, rps)])


def _make_edge_call(n_pad, wd, ch, with_den):
    mesh = plsc.VectorSubcoreMesh(core_axis_name="c", subcore_axis_name="s",
                                  num_cores=_NC, num_subcores=_NS)
    out_type = [jax.ShapeDtypeStruct((_NC, n_pad, wd), jnp.float32)]
    scratch = [
        pltpu.VMEM((ch, _B), jnp.int32),
        pltpu.VMEM((ch, _B), jnp.int32),
        pltpu.VMEM((_B, wd), jnp.float32),
        pltpu.VMEM((_B, wd), jnp.float32),
        pltpu.VMEM((_B, 16), jnp.float32),
        pltpu.VMEM((_B, 16), jnp.float32),
        pltpu.VMEM((_ZB, wd), jnp.float32),
    ]
    if with_den:
        out_type.append(jax.ShapeDtypeStruct((n_pad, 16), jnp.float32))
        scratch.append(pltpu.VMEM((_ZB, 16), jnp.float32))
    scratch.append(pltpu.VMEM_SHARED((n_pad, wd), jnp.float32))
    if with_den:
        scratch.append(pltpu.VMEM_SHARED((n_pad, 16), jnp.float32))
    scratch += [pltpu.SemaphoreType.DMA] * 4
    body = _edge_body_den if with_den else _edge_body_noden
    return pl.kernel(body, out_type=tuple(out_type), mesh=mesh,
                     scratch_types=scratch,
                     compiler_params=pltpu.CompilerParams(
                         use_tc_tiling_on_sc=False))


def _tc_ewb_body(w_ref, o_ref):
    o_ref[...] = jnp.broadcast_to(w_ref[...], o_ref.shape)


def _tc_pre_body(x_ref, wln_ref, bln_ref, wla_ref, wlb_ref, wr_ref, b_ref,
                 ga_ref, gb_ref, r_ref):
    lx = jnp.maximum(
        jnp.dot(x_ref[...], wln_ref[...], preferred_element_type=jnp.float32)
        + bln_ref[...], 0.0)
    ga_ref[...] = jnp.dot(lx, wla_ref[...], preferred_element_type=jnp.float32)
    gb_ref[...] = jnp.dot(lx, wlb_ref[...], preferred_element_type=jnp.float32)
    r_ref[...] = jnp.dot(lx, wr_ref[...],
                         preferred_element_type=jnp.float32) + b_ref[...]


def _tc_mid_body(ta_ref, tb_ref, d_ref, r_ref,
                 wln_ref, bln_ref, wla_ref, wlb_ref, wr_ref, b_ref,
                 ga_ref, gb_ref, r2_ref):
    den = jnp.maximum(d_ref[...][:, :1], 1e-6)
    t = jnp.concatenate([ta_ref[...], tb_ref[...]], axis=1)
    rst = t / den + r_ref[...]
    lx = jnp.maximum(
        jnp.dot(rst, wln_ref[...], preferred_element_type=jnp.float32)
        + bln_ref[...], 0.0)
    ga_ref[...] = jnp.dot(lx, wla_ref[...], preferred_element_type=jnp.float32)
    gb_ref[...] = jnp.dot(lx, wlb_ref[...], preferred_element_type=jnp.float32)
    r2_ref[...] = jnp.dot(lx, wr_ref[...],
                          preferred_element_type=jnp.float32) + b_ref[...]


def _tc_post_body(ta_ref, tb_ref, d_ref, r_ref, o_ref):
    den = jnp.maximum(d_ref[...][:, :1], 1e-6)
    t = jnp.concatenate([ta_ref[...], tb_ref[...]], axis=1)
    o_ref[...] = t / den + r_ref[...]


def _row_spec(rb, w):
    return pl.BlockSpec((rb, w), lambda i: (i, 0))


def _full_spec(shape):
    return pl.BlockSpec(shape, lambda i: tuple(0 for _ in shape))


def kernel(x, edge_index, edge_weight, W_ln1, b_ln1, W_l1, W_r1, b1,
           W_ln2, b_ln2, W_l2, W_r2, b2):
    n, d = x.shape
    h = W_l1.shape[1]
    cdim = W_l2.shape[1]
    e = edge_weight.shape[0]
    eps = e // _NS
    ch = eps // _B
    rb = 2000
    grid = (n // rb,)
    n_pad = ((n + _RPAD - 1) // _RPAD) * _RPAD

    src = edge_index[0].reshape(_NS, ch, _B)
    dst = edge_index[1].reshape(_NS, ch, _B)

    eb = 4000
    ewb = pl.pallas_call(
        _tc_ewb_body,
        grid=(e // eb,),
        in_specs=[pl.BlockSpec((eb, 1), lambda i: (i, 0))],
        out_specs=pl.BlockSpec((eb, 16), lambda i: (i, 0)),
        out_shape=jax.ShapeDtypeStruct((e, 16), jnp.float32),
    )(edge_weight.reshape(e, 1))
    ewb_r = ewb.reshape(_NS, ch, _B, 16)

    g1a, g1b, r1 = pl.pallas_call(
        _tc_pre_body,
        grid=grid,
        in_specs=[_row_spec(rb, d), _full_spec((d, d)), _full_spec((1, d)),
                  _full_spec((d, h // 2)), _full_spec((d, h // 2)),
                  _full_spec((d, h)), _full_spec((1, h))],
        out_specs=[_row_spec(rb, h // 2), _row_spec(rb, h // 2),
                   _row_spec(rb, h)],
        out_shape=[jax.ShapeDtypeStruct((n, h // 2), jnp.float32),
                   jax.ShapeDtypeStruct((n, h // 2), jnp.float32),
                   jax.ShapeDtypeStruct((n, h), jnp.float32)],
    )(x, W_ln1, b_ln1.reshape(1, d), W_l1[:, :h // 2], W_l1[:, h // 2:],
      W_r1, b1.reshape(1, h))

    g1s = jnp.stack([g1a, g1b])
    t1p, denp = _make_edge_call(n_pad, h // 2, ch, True)(src, dst, ewb_r, g1s)

    g2a, g2b, r2 = pl.pallas_call(
        _tc_mid_body,
        grid=grid,
        in_specs=[_row_spec(rb, h // 2), _row_spec(rb, h // 2),
                  _row_spec(rb, 16), _row_spec(rb, h),
                  _full_spec((h, h)), _full_spec((1, h)),
                  _full_spec((h, cdim // 2)), _full_spec((h, cdim // 2)),
                  _full_spec((h, cdim)), _full_spec((1, cdim))],
        out_specs=[_row_spec(rb, cdim // 2), _row_spec(rb, cdim // 2),
                   _row_spec(rb, cdim)],
        out_shape=[jax.ShapeDtypeStruct((n, cdim // 2), jnp.float32),
                   jax.ShapeDtypeStruct((n, cdim // 2), jnp.float32),
                   jax.ShapeDtypeStruct((n, cdim), jnp.float32)],
    )(t1p[0, :n], t1p[1, :n], denp[:n], r1, W_ln2, b_ln2.reshape(1, h),
      W_l2[:, :cdim // 2], W_l2[:, cdim // 2:], W_r2, b2.reshape(1, cdim))

    g2s = jnp.stack([g2a, g2b])
    (t2p,) = _make_edge_call(n_pad, cdim // 2, ch, False)(src, dst, ewb_r, g2s)

    out = pl.pallas_call(
        _tc_post_body,
        grid=grid,
        in_specs=[_row_spec(rb, cdim // 2), _row_spec(rb, cdim // 2),
                  _row_spec(rb, 16), _row_spec(rb, cdim)],
        out_specs=_row_spec(rb, cdim),
        out_shape=jax.ShapeDtypeStruct((n, cdim), jnp.float32),
    )(t2p[0, :n], t2p[1, :n], denp[:n], r2)
    return out


# speedup vs baseline: 4.1067x; 1.1060x over previous
import jax
import jax.numpy as jnp
from jax import lax
from jax.experimental import pallas as pl
from jax.experimental.pallas import tpu as pltpu
from jax.experimental.pallas import tpu_sc as plsc

_NC = 2
_NS = 16
_B = 80
_ZB = 128
_RPAD = _NS * _ZB


def _edge_body_den(src_h, dst_h, ewb_h, g_h, out_h, den_h,
                   src_v, dst_v, rows_a, rows_b, ewc_a, ewc_b,
                   zacc_v, zden_v, acc_sh, den_sh, sga, sgb, ssa, ssb):
    _edge_common(src_h, dst_h, ewb_h, g_h, out_h, den_h,
                 src_v, dst_v, rows_a, rows_b, ewc_a, ewc_b,
                 zacc_v, zden_v, acc_sh, den_sh, sga, sgb, ssa, ssb)


def _edge_body_noden(src_h, dst_h, ewb_h, g_h, out_h,
                     src_v, dst_v, rows_a, rows_b, ewc_a, ewc_b,
                     zacc_v, acc_sh, sga, sgb, ssa, ssb):
    _edge_common(src_h, dst_h, ewb_h, g_h, out_h, None,
                 src_v, dst_v, rows_a, rows_b, ewc_a, ewc_b,
                 zacc_v, None, acc_sh, None, sga, sgb, ssa, ssb)


def _edge_common(src_h, dst_h, ewb_h, g_h, out_h, den_h,
                 src_v, dst_v, rows_a, rows_b, ewc_a, ewc_b,
                 zacc_v, zden_v, acc_sh, den_sh, sga, sgb, ssa, ssb):
    n_pad, wd = acc_sh.shape
    ch, b = src_v.shape
    rps = n_pad // _NS
    zn = rps // _ZB
    c = lax.axis_index("c")
    s = lax.axis_index("s")

    def zrow(i, carry):
        for cc in range(wd // 16):
            zacc_v[i, pl.ds(cc * 16, 16)] = jnp.zeros((16,), jnp.float32)
        if zden_v is not None:
            zden_v[i, :] = jnp.zeros((16,), jnp.float32)
        return carry

    lax.fori_loop(0, _ZB, zrow, 0)
    for z in range(zn):
        row0 = s * rps + z * _ZB
        pltpu.sync_copy(zacc_v, acc_sh.at[pl.ds(row0, _ZB)])
        if den_sh is not None:
            pltpu.sync_copy(zden_v, den_sh.at[pl.ds(row0, _ZB)])
    plsc.subcore_barrier()

    pltpu.sync_copy(src_h.at[s], src_v)
    pltpu.sync_copy(dst_h.at[s], dst_v)

    def start_gather(j, rows_v, ewc_v, sem):
        pltpu.async_copy(g_h.at[c].at[src_v.at[j]], rows_v, sem)
        pltpu.async_copy(ewb_h.at[s, j], ewc_v, sem)

    def wait_gather(j, rows_v, ewc_v, sem):
        pltpu.make_async_copy(g_h.at[c].at[src_v.at[j]], rows_v, sem).wait()
        pltpu.make_async_copy(ewb_h.at[s, j], ewc_v, sem).wait()

    def scale(rows_v, ewc_v):
        def grp(g8, carry):
            for i in range(8):
                e = g8 * 8 + i
                wv = ewc_v[e, :]
                for cc in range(wd // 16):
                    sl = pl.ds(cc * 16, 16)
                    rows_v[e, sl] = rows_v[e, sl] * wv
            return carry
        lax.fori_loop(0, b // 8, grp, 0)

    def proc(j, rows_v, ewc_v):
        scale(rows_v, ewc_v)
        pltpu.sync_copy(rows_v, acc_sh.at[dst_v.at[j]], add=True)
        if den_sh is not None:
            @pl.when(c == 0)
            def _():
                pltpu.sync_copy(ewc_v, den_sh.at[dst_v.at[j]], add=True)

    start_gather(0, rows_a, ewc_a, sga)
    start_gather(1, rows_b, ewc_b, sgb)

    def pair(p, carry):
        j = 2 * p
        wait_gather(j, rows_a, ewc_a, sga)
        proc(j, rows_a, ewc_a)

        @pl.when(j + 2 < ch)
        def _():
            start_gather(j + 2, rows_a, ewc_a, sga)

        wait_gather(j + 1, rows_b, ewc_b, sgb)
        proc(j + 1, rows_b, ewc_b)

        @pl.when(j + 3 < ch)
        def _():
            start_gather(j + 3, rows_b, ewc_b, sgb)

        return carry

    lax.fori_loop(0, ch // 2, pair, 0)
    plsc.subcore_barrier()

    row0 = s * rps
    pltpu.sync_copy(acc_sh.at[pl.ds(row0, rps)], out_h.at[c, pl.ds(row0, rps)])
    if den_sh is not None:
        @pl.when(c == 0)
        def _():
            pltpu.sync_copy(den_sh.at[pl.ds(row0, rps)],
                            den_h.at[pl.ds(row0, rps)])


def _make_edge_call(n_pad, wd, ch, with_den):
    mesh = plsc.VectorSubcoreMesh(core_axis_name="c", subcore_axis_name="s",
                                  num_cores=_NC, num_subcores=_NS)
    out_type = [jax.ShapeDtypeStruct((_NC, n_pad, wd), jnp.float32)]
    scratch = [
        pltpu.VMEM((ch, _B), jnp.int32),
        pltpu.VMEM((ch, _B), jnp.int32),
        pltpu.VMEM((_B, wd), jnp.float32),
        pltpu.VMEM((_B, wd), jnp.float32),
        pltpu.VMEM((_B, 16), jnp.float32),
        pltpu.VMEM((_B, 16), jnp.float32),
        pltpu.VMEM((_ZB, wd), jnp.float32),
    ]
    if with_den:
        out_type.append(jax.ShapeDtypeStruct((n_pad, 16), jnp.float32))
        scratch.append(pltpu.VMEM((_ZB, 16), jnp.float32))
    scratch.append(pltpu.VMEM_SHARED((n_pad, wd), jnp.float32))
    if with_den:
        scratch.append(pltpu.VMEM_SHARED((n_pad, 16), jnp.float32))
    scratch += [pltpu.SemaphoreType.DMA] * 4
    body = _edge_body_den if with_den else _edge_body_noden
    return pl.kernel(body, out_type=tuple(out_type), mesh=mesh,
                     scratch_types=scratch,
                     compiler_params=pltpu.CompilerParams(
                         use_tc_tiling_on_sc=False))


def _tc_ewb_body(w_ref, o_ref):
    o_ref[...] = jnp.broadcast_to(w_ref[...], o_ref.shape)


def _tc_pre_body(x_ref, wln_ref, bln_ref, wla_ref, wlb_ref, wr_ref, b_ref,
                 ga_ref, gb_ref, r_ref):
    lx = jnp.maximum(
        jnp.dot(x_ref[...], wln_ref[...], preferred_element_type=jnp.float32)
        + bln_ref[...], 0.0)
    ga_ref[...] = jnp.dot(lx, wla_ref[...], preferred_element_type=jnp.float32)
    gb_ref[...] = jnp.dot(lx, wlb_ref[...], preferred_element_type=jnp.float32)
    r_ref[...] = jnp.dot(lx, wr_ref[...],
                         preferred_element_type=jnp.float32) + b_ref[...]


def _tc_mid_body(ta_ref, tb_ref, d_ref, r_ref,
                 wln_ref, bln_ref, wla_ref, wlb_ref, wr_ref, b_ref,
                 ga_ref, gb_ref, r2_ref):
    den = jnp.maximum(d_ref[...][:, :1], 1e-6)
    t = jnp.concatenate([ta_ref[...], tb_ref[...]], axis=1)
    rst = t / den + r_ref[...]
    lx = jnp.maximum(
        jnp.dot(rst, wln_ref[...], preferred_element_type=jnp.float32)
        + bln_ref[...], 0.0)
    ga_ref[...] = jnp.dot(lx, wla_ref[...], preferred_element_type=jnp.float32)
    gb_ref[...] = jnp.dot(lx, wlb_ref[...], preferred_element_type=jnp.float32)
    r2_ref[...] = jnp.dot(lx, wr_ref[...],
                          preferred_element_type=jnp.float32) + b_ref[...]


def _tc_post_body(ta_ref, tb_ref, d_ref, r_ref, o_ref):
    den = jnp.maximum(d_ref[...][:, :1], 1e-6)
    t = jnp.concatenate([ta_ref[...], tb_ref[...]], axis=1)
    o_ref[...] = t / den + r_ref[...]


def _row_spec(rb, w):
    return pl.BlockSpec((rb, w), lambda i: (i, 0))


def _full_spec(shape):
    return pl.BlockSpec(shape, lambda i: tuple(0 for _ in shape))


def kernel(x, edge_index, edge_weight, W_ln1, b_ln1, W_l1, W_r1, b1,
           W_ln2, b_ln2, W_l2, W_r2, b2):
    n, d = x.shape
    h = W_l1.shape[1]
    cdim = W_l2.shape[1]
    e = edge_weight.shape[0]
    eps = e // _NS
    ch = eps // _B
    rb = 2000
    grid = (n // rb,)
    n_pad = ((n + _RPAD - 1) // _RPAD) * _RPAD

    src = edge_index[0].reshape(_NS, ch, _B)
    dst = edge_index[1].reshape(_NS, ch, _B)

    eb = 4000
    ewb = pl.pallas_call(
        _tc_ewb_body,
        grid=(e // eb,),
        in_specs=[pl.BlockSpec((eb, 1), lambda i: (i, 0))],
        out_specs=pl.BlockSpec((eb, 16), lambda i: (i, 0)),
        out_shape=jax.ShapeDtypeStruct((e, 16), jnp.float32),
    )(edge_weight.reshape(e, 1))
    ewb_r = ewb.reshape(_NS, ch, _B, 16)

    g1a, g1b, r1 = pl.pallas_call(
        _tc_pre_body,
        grid=grid,
        in_specs=[_row_spec(rb, d), _full_spec((d, d)), _full_spec((1, d)),
                  _full_spec((d, h // 2)), _full_spec((d, h // 2)),
                  _full_spec((d, h)), _full_spec((1, h))],
        out_specs=[_row_spec(rb, h // 2), _row_spec(rb, h // 2),
                   _row_spec(rb, h)],
        out_shape=[jax.ShapeDtypeStruct((n, h // 2), jnp.float32),
                   jax.ShapeDtypeStruct((n, h // 2), jnp.float32),
                   jax.ShapeDtypeStruct((n, h), jnp.float32)],
    )(x, W_ln1, b_ln1.reshape(1, d), W_l1[:, :h // 2], W_l1[:, h // 2:],
      W_r1, b1.reshape(1, h))

    g1s = jnp.stack([g1a, g1b])
    t1p, denp = _make_edge_call(n_pad, h // 2, ch, True)(src, dst, ewb_r, g1s)

    g2a, g2b, r2 = pl.pallas_call(
        _tc_mid_body,
        grid=grid,
        in_specs=[_row_spec(rb, h // 2), _row_spec(rb, h // 2),
                  _row_spec(rb, 16), _row_spec(rb, h),
                  _full_spec((h, h)), _full_spec((1, h)),
                  _full_spec((h, cdim // 2)), _full_spec((h, cdim // 2)),
                  _full_spec((h, cdim)), _full_spec((1, cdim))],
        out_specs=[_row_spec(rb, cdim // 2), _row_spec(rb, cdim // 2),
                   _row_spec(rb, cdim)],
        out_shape=[jax.ShapeDtypeStruct((n, cdim // 2), jnp.float32),
                   jax.ShapeDtypeStruct((n, cdim // 2), jnp.float32),
                   jax.ShapeDtypeStruct((n, cdim), jnp.float32)],
    )(t1p[0, :n], t1p[1, :n], denp[:n], r1, W_ln2, b_ln2.reshape(1, h),
      W_l2[:, :cdim // 2], W_l2[:, cdim // 2:], W_r2, b2.reshape(1, cdim))

    g2s = jnp.stack([g2a, g2b])
    (t2p,) = _make_edge_call(n_pad, cdim // 2, ch, False)(src, dst, ewb_r, g2s)

    out = pl.pallas_call(
        _tc_post_body,
        grid=grid,
        in_specs=[_row_spec(rb, cdim // 2), _row_spec(rb, cdim // 2),
                  _row_spec(rb, 16), _row_spec(rb, cdim)],
        out_specs=_row_spec(rb, cdim),
        out_shape=jax.ShapeDtypeStruct((n, cdim), jnp.float32),
    )(t2p[0, :n], t2p[1, :n], denp[:n], r2)
    return out
